# baseline (speedup 1.0000x reference)
import functools
import jax
import jax.numpy as jnp
from jax import lax
from jax.experimental import pallas as pl
from jax.experimental.pallas import tpu as pltpu


def _pick_block(n, target):
    if n % target == 0:
        return target
    return n


def _proj_kernel(h_ref, wT_ref, b_ref, o_ref):
    o_ref[...] = jax.nn.relu(
        jnp.dot(h_ref[...], wT_ref[...], preferred_element_type=jnp.float32)
        + b_ref[...]
    )


def _project_nodes(h, proj_WT, proj_b):
    n, d_in = h.shape
    d_out = proj_WT.shape[1]
    bn = _pick_block(n, 2000)
    return pl.pallas_call(
        _proj_kernel,
        grid=(n // bn,),
        in_specs=[
            pl.BlockSpec((bn, d_in), lambda i: (i, 0)),
            pl.BlockSpec((d_in, d_out), lambda i: (0, 0)),
            pl.BlockSpec((1, d_out), lambda i: (0, 0)),
        ],
        out_specs=pl.BlockSpec((bn, d_out), lambda i: (i, 0)),
        out_shape=jax.ShapeDtypeStruct((n, d_out), jnp.float32),
    )(h, proj_WT, proj_b.reshape(1, d_out))


def _msg_kernel(e_ref, hs_ref, w1T_ref, b1_ref, tflat_ref, b2m_ref, m_ref):
    f = jax.nn.relu(
        jnp.dot(e_ref[...], w1T_ref[...], preferred_element_type=jnp.float32)
        + b1_ref[...]
    )
    hs = hs_ref[...]
    b, d_out = hs.shape
    d_eh = f.shape[1]
    outer = (hs[:, :, None] * f[:, None, :]).reshape(b, d_out * d_eh)
    m = jnp.dot(outer, tflat_ref[...], preferred_element_type=jnp.float32)
    m = m + jnp.dot(hs, b2m_ref[...], preferred_element_type=jnp.float32)
    m_ref[...] = m


def _messages(e, hs, w1T, b1, tflat, b2m):
    ecnt, d_e = e.shape
    d_out = hs.shape[1]
    d_eh = w1T.shape[1]
    be = _pick_block(ecnt, 256)
    return pl.pallas_call(
        _msg_kernel,
        grid=(ecnt // be,),
        in_specs=[
            pl.BlockSpec((be, d_e), lambda i: (i, 0)),
            pl.BlockSpec((be, d_out), lambda i: (i, 0)),
            pl.BlockSpec((d_e, d_eh), lambda i: (0, 0)),
            pl.BlockSpec((1, d_eh), lambda i: (0, 0)),
            pl.BlockSpec((d_out * d_eh, d_out), lambda i: (0, 0)),
            pl.BlockSpec((d_out, d_out), lambda i: (0, 0)),
        ],
        out_specs=pl.BlockSpec((be, d_out), lambda i: (i, 0)),
        out_shape=jax.ShapeDtypeStruct((ecnt, d_out), jnp.float32),
    )(e, hs, w1T, b1.reshape(1, d_eh), tflat, b2m)


def _gru_kernel(agg_ref, ht_ref, gb_ref,
                wr_i_ref, wz_i_ref, wn_i_ref,
                wr_h_ref, wz_h_ref, wn_h_ref,
                br_i_ref, bz_i_ref, bn_i_ref,
                br_h_ref, bz_h_ref, bn_h_ref,
                o_ref):
    a = jax.nn.relu(agg_ref[...] + gb_ref[...])
    ht = ht_ref[...]
    dot = functools.partial(jnp.dot, preferred_element_type=jnp.float32)
    r = jax.nn.sigmoid(dot(a, wr_i_ref[...]) + br_i_ref[...]
                       + dot(ht, wr_h_ref[...]) + br_h_ref[...])
    z = jax.nn.sigmoid(dot(a, wz_i_ref[...]) + bz_i_ref[...]
                       + dot(ht, wz_h_ref[...]) + bz_h_ref[...])
    nn = jnp.tanh(dot(a, wn_i_ref[...]) + bn_i_ref[...]
                  + r * (dot(ht, wn_h_ref[...]) + bn_h_ref[...]))
    o_ref[...] = (1.0 - z) * nn + z * ht


def _gru_update(agg, ht, gnn_b, gru_ws, gru_bs):
    n, d_out = agg.shape
    bn = _pick_block(n, 2000)
    mat_spec = pl.BlockSpec((d_out, d_out), lambda i: (0, 0))
    vec_spec = pl.BlockSpec((1, d_out), lambda i: (0, 0))
    return pl.pallas_call(
        _gru_kernel,
        grid=(n // bn,),
        in_specs=[
            pl.BlockSpec((bn, d_out), lambda i: (i, 0)),
            pl.BlockSpec((bn, d_out), lambda i: (i, 0)),
            vec_spec,
            mat_spec, mat_spec, mat_spec,
            mat_spec, mat_spec, mat_spec,
            vec_spec, vec_spec, vec_spec,
            vec_spec, vec_spec, vec_spec,
        ],
        out_specs=pl.BlockSpec((bn, d_out), lambda i: (i, 0)),
        out_shape=jax.ShapeDtypeStruct((n, d_out), jnp.float32),
    )(agg, ht, gnn_b.reshape(1, d_out), *gru_ws, *gru_bs)


def kernel(h, e, edge_index, proj_W, proj_b, edge_W1, edge_b1, edge_W2,
           edge_b2, gnn_b, W_ih, W_hh, b_ih, b_hh):
    n, d_in = h.shape
    ecnt, d_e = e.shape
    d_out = proj_W.shape[0]
    d_eh = edge_W1.shape[0]
    steps = 3

    src = edge_index[0]
    dst = edge_index[1]

    proj_WT = proj_W.T
    w1T = edge_W1.T
    tflat = (edge_W2.reshape(d_out, d_out, d_eh)
             .transpose(0, 2, 1)
             .reshape(d_out * d_eh, d_out))
    b2m = edge_b2.reshape(d_out, d_out)
    wih = W_ih.reshape(3, d_out, d_out)
    whh = W_hh.reshape(3, d_out, d_out)
    gru_ws = (wih[0].T, wih[1].T, wih[2].T, whh[0].T, whh[1].T, whh[2].T)
    bih = b_ih.reshape(3, 1, d_out)
    bhh = b_hh.reshape(3, 1, d_out)
    gru_bs = (bih[0], bih[1], bih[2], bhh[0], bhh[1], bhh[2])

    hcur = _project_nodes(h, proj_WT, proj_b)
    ht = hcur
    for _ in range(steps):
        hs = jnp.take(hcur, src, axis=0)
        m = _messages(e, hs, w1T, edge_b1, tflat, b2m)
        agg = jax.ops.segment_sum(m, dst, num_segments=n)
        hcur = _gru_update(agg, ht, gnn_b, gru_ws, gru_bs)
        ht = hcur
    return hcur


# speedup vs baseline: 1.8149x; 1.8149x over previous
import functools
import jax
import jax.numpy as jnp
from jax import lax
from jax.experimental import pallas as pl
from jax.experimental.pallas import tpu as pltpu


def _pick_block(n, target):
    if n % target == 0:
        return target
    return n


def _proj_kernel(h_ref, wT_ref, b_ref, o_ref):
    o_ref[...] = jax.nn.relu(
        jnp.dot(h_ref[...], wT_ref[...], preferred_element_type=jnp.float32)
        + b_ref[...]
    )


def _project_nodes(h, proj_WT, proj_b):
    n, d_in = h.shape
    d_out = proj_WT.shape[1]
    bn = _pick_block(n, 2000)
    return pl.pallas_call(
        _proj_kernel,
        grid=(n // bn,),
        in_specs=[
            pl.BlockSpec((bn, d_in), lambda i: (i, 0)),
            pl.BlockSpec((d_in, d_out), lambda i: (0, 0)),
            pl.BlockSpec((1, d_out), lambda i: (0, 0)),
        ],
        out_specs=pl.BlockSpec((bn, d_out), lambda i: (i, 0)),
        out_shape=jax.ShapeDtypeStruct((n, d_out), jnp.float32),
    )(h, proj_WT, proj_b.reshape(1, d_out))


def _msg_kernel(eT_ref, hs_ref, w1x_ref, t2_ref, b2mT_ref, m_ref):
    fT = jax.nn.relu(
        jnp.dot(w1x_ref[...], eT_ref[...], preferred_element_type=jnp.float32)
    )
    hsT = hs_ref[...].T
    d_out, b = hsT.shape
    d_eh = fT.shape[0]
    outerT = (hsT[:, None, :] * fT[None, :, :]).reshape(d_out * d_eh, b)
    mT = jnp.dot(t2_ref[...], outerT, preferred_element_type=jnp.float32)
    mT = mT + jnp.dot(b2mT_ref[...], hsT, preferred_element_type=jnp.float32)
    m_ref[...] = mT.T


def _messages(eT, hs, w1x, t2, b2mT):
    d_e1, ecnt = eT.shape
    d_out = hs.shape[1]
    d_eh = w1x.shape[0]
    be = _pick_block(ecnt, 256)
    return pl.pallas_call(
        _msg_kernel,
        grid=(ecnt // be,),
        in_specs=[
            pl.BlockSpec((d_e1, be), lambda i: (0, i)),
            pl.BlockSpec((be, d_out), lambda i: (i, 0)),
            pl.BlockSpec((d_eh, d_e1), lambda i: (0, 0)),
            pl.BlockSpec((d_out, d_out * d_eh), lambda i: (0, 0)),
            pl.BlockSpec((d_out, d_out), lambda i: (0, 0)),
        ],
        out_specs=pl.BlockSpec((be, d_out), lambda i: (i, 0)),
        out_shape=jax.ShapeDtypeStruct((ecnt, d_out), jnp.float32),
    )(eT, hs, w1x, t2, b2mT)


def _gru_kernel(agg_ref, ht_ref, gb_ref,
                wr_i_ref, wz_i_ref, wn_i_ref,
                wr_h_ref, wz_h_ref, wn_h_ref,
                br_i_ref, bz_i_ref, bn_i_ref,
                br_h_ref, bz_h_ref, bn_h_ref,
                o_ref):
    a = jax.nn.relu(agg_ref[...] + gb_ref[...])
    ht = ht_ref[...]
    dot = functools.partial(jnp.dot, preferred_element_type=jnp.float32)
    r = jax.nn.sigmoid(dot(a, wr_i_ref[...]) + br_i_ref[...]
                       + dot(ht, wr_h_ref[...]) + br_h_ref[...])
    z = jax.nn.sigmoid(dot(a, wz_i_ref[...]) + bz_i_ref[...]
                       + dot(ht, wz_h_ref[...]) + bz_h_ref[...])
    nn = jnp.tanh(dot(a, wn_i_ref[...]) + bn_i_ref[...]
                  + r * (dot(ht, wn_h_ref[...]) + bn_h_ref[...]))
    o_ref[...] = (1.0 - z) * nn + z * ht


def _gru_update(agg, ht, gnn_b, gru_ws, gru_bs):
    n, d_out = agg.shape
    bn = _pick_block(n, 2000)
    mat_spec = pl.BlockSpec((d_out, d_out), lambda i: (0, 0))
    vec_spec = pl.BlockSpec((1, d_out), lambda i: (0, 0))
    return pl.pallas_call(
        _gru_kernel,
        grid=(n // bn,),
        in_specs=[
            pl.BlockSpec((bn, d_out), lambda i: (i, 0)),
            pl.BlockSpec((bn, d_out), lambda i: (i, 0)),
            vec_spec,
            mat_spec, mat_spec, mat_spec,
            mat_spec, mat_spec, mat_spec,
            vec_spec, vec_spec, vec_spec,
            vec_spec, vec_spec, vec_spec,
        ],
        out_specs=pl.BlockSpec((bn, d_out), lambda i: (i, 0)),
        out_shape=jax.ShapeDtypeStruct((n, d_out), jnp.float32),
    )(agg, ht, gnn_b.reshape(1, d_out), *gru_ws, *gru_bs)


def kernel(h, e, edge_index, proj_W, proj_b, edge_W1, edge_b1, edge_W2,
           edge_b2, gnn_b, W_ih, W_hh, b_ih, b_hh):
    n, d_in = h.shape
    ecnt, d_e = e.shape
    d_out = proj_W.shape[0]
    d_eh = edge_W1.shape[0]
    steps = 3

    src = edge_index[0]
    dst = edge_index[1]

    proj_WT = proj_W.T
    eT = jnp.concatenate(
        [e.T, jnp.ones((1, ecnt), jnp.float32)], axis=0)
    w1x = jnp.concatenate([edge_W1, edge_b1[:, None]], axis=1)
    t2 = (edge_W2.reshape(d_out, d_out, d_eh)
          .transpose(1, 0, 2)
          .reshape(d_out, d_out * d_eh))
    b2mT = edge_b2.reshape(d_out, d_out).T
    wih = W_ih.reshape(3, d_out, d_out)
    whh = W_hh.reshape(3, d_out, d_out)
    gru_ws = (wih[0].T, wih[1].T, wih[2].T, whh[0].T, whh[1].T, whh[2].T)
    bih = b_ih.reshape(3, 1, d_out)
    bhh = b_hh.reshape(3, 1, d_out)
    gru_bs = (bih[0], bih[1], bih[2], bhh[0], bhh[1], bhh[2])

    hcur = _project_nodes(h, proj_WT, proj_b)
    ht = hcur
    for _ in range(steps):
        hs = jnp.take(hcur, src, axis=0)
        m = _messages(eT, hs, w1x, t2, b2mT)
        agg = jax.ops.segment_sum(m, dst, num_segments=n)
        hcur = _gru_update(agg, ht, gnn_b, gru_ws, gru_bs)
        ht = hcur
    return hcur


# speedup vs baseline: 2.6156x; 1.4412x over previous
import functools
import jax
import jax.numpy as jnp
from jax import lax
from jax.experimental import pallas as pl
from jax.experimental.pallas import tpu as pltpu
from jax.experimental.pallas import tpu_sc as plsc

_NC = 2
_NS = 16
_NW = _NC * _NS


_CH = 128


def _sc_gather(table, idx):
    ecnt = idx.shape[0]
    d = table.shape[1]
    bpw = ecnt // _NW
    n_ch = bpw // _CH
    tail = bpw - n_ch * _CH
    mesh = plsc.VectorSubcoreMesh(core_axis_name="c", subcore_axis_name="s")

    @functools.partial(
        pl.kernel, mesh=mesh,
        out_type=jax.ShapeDtypeStruct((ecnt, d), jnp.float32),
        compiler_params=pltpu.CompilerParams(use_tc_tiling_on_sc=False),
        scratch_types=[
            pltpu.VMEM((_CH,), jnp.int32),
            pltpu.VMEM((_CH, d), jnp.float32),
            pltpu.VMEM((max(tail, 1),), jnp.int32),
            pltpu.VMEM((max(tail, 1), d), jnp.float32),
            pltpu.SemaphoreType.DMA,
        ],
    )
    def k(table_hbm, idx_hbm, out_hbm, idx_v, rows_v, idx_t, rows_t, sem):
        wid = lax.axis_index("s") * _NC + lax.axis_index("c")
        base = wid * bpw

        def body(j, carry):
            off = base + j * _CH
            pltpu.sync_copy(idx_hbm.at[pl.ds(off, _CH)], idx_v)
            pltpu.async_copy(table_hbm.at[idx_v], rows_v, sem).wait()
            pltpu.sync_copy(rows_v, out_hbm.at[pl.ds(off, _CH)])
            return carry

        lax.fori_loop(0, n_ch, body, 0)
        if tail:
            off = base + n_ch * _CH
            pltpu.sync_copy(idx_hbm.at[pl.ds(off, tail)], idx_t)
            pltpu.async_copy(table_hbm.at[idx_t], rows_t, sem).wait()
            pltpu.sync_copy(rows_t, out_hbm.at[pl.ds(off, tail)])

    return k(table, idx)


def _sc_scatter_add(m, dst, npad, zeros_hbm):
    ecnt, d = m.shape
    bpw = ecnt // _NW
    n_ch = bpw // _CH
    tail = bpw - n_ch * _CH
    rows_per_tile = npad // _NS
    mesh = plsc.VectorSubcoreMesh(core_axis_name="c", subcore_axis_name="s")

    @functools.partial(
        pl.kernel, mesh=mesh,
        out_type=jax.ShapeDtypeStruct((_NC * npad, d), jnp.float32),
        compiler_params=pltpu.CompilerParams(use_tc_tiling_on_sc=False),
        scratch_types=[
            pltpu.VMEM((_CH,), jnp.int32),
            pltpu.VMEM((_CH, d), jnp.float32),
            pltpu.VMEM((max(tail, 1),), jnp.int32),
            pltpu.VMEM((max(tail, 1), d), jnp.float32),
            pltpu.VMEM_SHARED((npad, d), jnp.float32),
        ],
    )
    def k(m_hbm, dst_hbm, z_hbm, out_hbm, idx_v, rows_v, idx_t, rows_t, acc):
        cid = lax.axis_index("c")
        sid = lax.axis_index("s")
        wid = sid * _NC + cid
        base = wid * bpw
        trow = sid * rows_per_tile
        pltpu.sync_copy(z_hbm.at[pl.ds(trow, rows_per_tile)],
                        acc.at[pl.ds(trow, rows_per_tile)])
        plsc.subcore_barrier()

        def body(j, carry):
            off = base + j * _CH
            pltpu.sync_copy(dst_hbm.at[pl.ds(off, _CH)], idx_v)
            pltpu.sync_copy(m_hbm.at[pl.ds(off, _CH)], rows_v)
            pltpu.sync_copy(rows_v, acc.at[idx_v], add=True)
            return carry

        lax.fori_loop(0, n_ch, body, 0)
        if tail:
            off = base + n_ch * _CH
            pltpu.sync_copy(dst_hbm.at[pl.ds(off, tail)], idx_t)
            pltpu.sync_copy(m_hbm.at[pl.ds(off, tail)], rows_t)
            pltpu.sync_copy(rows_t, acc.at[idx_t], add=True)
        plsc.subcore_barrier()
        pltpu.sync_copy(acc.at[pl.ds(trow, rows_per_tile)],
                        out_hbm.at[pl.ds(cid * npad + trow, rows_per_tile)])

    return k(m, dst, zeros_hbm).reshape(_NC, npad, d)


def _pick_block(n, target):
    if n % target == 0:
        return target
    return n


def _proj_kernel(h_ref, wT_ref, b_ref, o_ref):
    o_ref[...] = jax.nn.relu(
        jnp.dot(h_ref[...], wT_ref[...], preferred_element_type=jnp.float32)
        + b_ref[...]
    )


def _project_nodes(h, proj_WT, proj_b):
    n, d_in = h.shape
    d_out = proj_WT.shape[1]
    bn = _pick_block(n, 2000)
    return pl.pallas_call(
        _proj_kernel,
        grid=(n // bn,),
        in_specs=[
            pl.BlockSpec((bn, d_in), lambda i: (i, 0)),
            pl.BlockSpec((d_in, d_out), lambda i: (0, 0)),
            pl.BlockSpec((1, d_out), lambda i: (0, 0)),
        ],
        out_specs=pl.BlockSpec((bn, d_out), lambda i: (i, 0)),
        out_shape=jax.ShapeDtypeStruct((n, d_out), jnp.float32),
    )(h, proj_WT, proj_b.reshape(1, d_out))


def _msg_kernel(eT_ref, hs_ref, w1x_ref, t2_ref, b2mT_ref, m_ref):
    fT = jax.nn.relu(
        jnp.dot(w1x_ref[...], eT_ref[...], preferred_element_type=jnp.float32)
    )
    hsT = hs_ref[...].T
    d_out, b = hsT.shape
    d_eh = fT.shape[0]
    outerT = (hsT[:, None, :] * fT[None, :, :]).reshape(d_out * d_eh, b)
    mT = jnp.dot(t2_ref[...], outerT, preferred_element_type=jnp.float32)
    mT = mT + jnp.dot(b2mT_ref[...], hsT, preferred_element_type=jnp.float32)
    m_ref[...] = mT.T


def _messages(eT, hs, w1x, t2, b2mT):
    d_e1, ecnt = eT.shape
    d_out = hs.shape[1]
    d_eh = w1x.shape[0]
    be = _pick_block(ecnt, 256)
    return pl.pallas_call(
        _msg_kernel,
        grid=(ecnt // be,),
        in_specs=[
            pl.BlockSpec((d_e1, be), lambda i: (0, i)),
            pl.BlockSpec((be, d_out), lambda i: (i, 0)),
            pl.BlockSpec((d_eh, d_e1), lambda i: (0, 0)),
            pl.BlockSpec((d_out, d_out * d_eh), lambda i: (0, 0)),
            pl.BlockSpec((d_out, d_out), lambda i: (0, 0)),
        ],
        out_specs=pl.BlockSpec((be, d_out), lambda i: (i, 0)),
        out_shape=jax.ShapeDtypeStruct((ecnt, d_out), jnp.float32),
    )(eT, hs, w1x, t2, b2mT)


def _gru_kernel(pp_ref, ht_ref, gb_ref,
                wr_i_ref, wz_i_ref, wn_i_ref,
                wr_h_ref, wz_h_ref, wn_h_ref,
                br_i_ref, bz_i_ref, bn_i_ref,
                br_h_ref, bz_h_ref, bn_h_ref,
                o_ref):
    a = jax.nn.relu(pp_ref[0] + pp_ref[1] + gb_ref[...])
    ht = ht_ref[...]
    dot = functools.partial(jnp.dot, preferred_element_type=jnp.float32)
    r = jax.nn.sigmoid(dot(a, wr_i_ref[...]) + br_i_ref[...]
                       + dot(ht, wr_h_ref[...]) + br_h_ref[...])
    z = jax.nn.sigmoid(dot(a, wz_i_ref[...]) + bz_i_ref[...]
                       + dot(ht, wz_h_ref[...]) + bz_h_ref[...])
    nn = jnp.tanh(dot(a, wn_i_ref[...]) + bn_i_ref[...]
                  + r * (dot(ht, wn_h_ref[...]) + bn_h_ref[...]))
    o_ref[...] = (1.0 - z) * nn + z * ht


def _gru_update(partials, ht, gnn_b, gru_ws, gru_bs):
    n, d_out = ht.shape
    bn = _pick_block(n, 2000)
    mat_spec = pl.BlockSpec((d_out, d_out), lambda i: (0, 0))
    vec_spec = pl.BlockSpec((1, d_out), lambda i: (0, 0))
    return pl.pallas_call(
        _gru_kernel,
        grid=(n // bn,),
        in_specs=[
            pl.BlockSpec((partials.shape[0], bn, d_out), lambda i: (0, i, 0)),
            pl.BlockSpec((bn, d_out), lambda i: (i, 0)),
            vec_spec,
            mat_spec, mat_spec, mat_spec,
            mat_spec, mat_spec, mat_spec,
            vec_spec, vec_spec, vec_spec,
            vec_spec, vec_spec, vec_spec,
        ],
        out_specs=pl.BlockSpec((bn, d_out), lambda i: (i, 0)),
        out_shape=jax.ShapeDtypeStruct((n, d_out), jnp.float32),
    )(partials, ht, gnn_b.reshape(1, d_out), *gru_ws, *gru_bs)


def kernel(h, e, edge_index, proj_W, proj_b, edge_W1, edge_b1, edge_W2,
           edge_b2, gnn_b, W_ih, W_hh, b_ih, b_hh):
    n, d_in = h.shape
    ecnt, d_e = e.shape
    d_out = proj_W.shape[0]
    d_eh = edge_W1.shape[0]
    steps = 3

    src = edge_index[0]
    dst = edge_index[1]

    proj_WT = proj_W.T
    eT = jnp.concatenate(
        [e.T, jnp.ones((1, ecnt), jnp.float32)], axis=0)
    w1x = jnp.concatenate([edge_W1, edge_b1[:, None]], axis=1)
    t2 = (edge_W2.reshape(d_out, d_out, d_eh)
          .transpose(1, 0, 2)
          .reshape(d_out, d_out * d_eh))
    b2mT = edge_b2.reshape(d_out, d_out).T
    wih = W_ih.reshape(3, d_out, d_out)
    whh = W_hh.reshape(3, d_out, d_out)
    gru_ws = (wih[0].T, wih[1].T, wih[2].T, whh[0].T, whh[1].T, whh[2].T)
    bih = b_ih.reshape(3, 1, d_out)
    bhh = b_hh.reshape(3, 1, d_out)
    gru_bs = (bih[0], bih[1], bih[2], bhh[0], bhh[1], bhh[2])

    npad = ((n + 16 * _NS - 1) // (16 * _NS)) * (16 * _NS)
    zeros_hbm = jnp.zeros((npad, d_out), jnp.float32)

    hcur = _project_nodes(h, proj_WT, proj_b)
    ht = hcur
    for _ in range(steps):
        hs = _sc_gather(hcur, src)
        m = _messages(eT, hs, w1x, t2, b2mT)
        partials = _sc_scatter_add(m, dst, npad, zeros_hbm)
        hcur = _gru_update(partials, ht, gnn_b, gru_ws, gru_bs)
        ht = hcur
    return hcur


# speedup vs baseline: 3.2692x; 1.2499x over previous
import functools
import jax
import jax.numpy as jnp
from jax import lax
from jax.experimental import pallas as pl
from jax.experimental.pallas import tpu as pltpu
from jax.experimental.pallas import tpu_sc as plsc

_NC = 2
_NS = 16
_NW = _NC * _NS


_CH = 128


def _sc_gather(table, idx):
    ecnt = idx.shape[0]
    d = table.shape[1]
    bpw = ecnt // _NW
    n_ch = bpw // _CH
    tail = bpw - n_ch * _CH
    mesh = plsc.VectorSubcoreMesh(core_axis_name="c", subcore_axis_name="s")

    @functools.partial(
        pl.kernel, mesh=mesh,
        out_type=jax.ShapeDtypeStruct((ecnt, d), jnp.float32),
        compiler_params=pltpu.CompilerParams(use_tc_tiling_on_sc=False),
        scratch_types=[
            pltpu.VMEM((_CH,), jnp.int32),
            pltpu.VMEM((_CH, d), jnp.float32),
            pltpu.VMEM((max(tail, 1),), jnp.int32),
            pltpu.VMEM((max(tail, 1), d), jnp.float32),
            pltpu.SemaphoreType.DMA,
        ],
    )
    def k(table_hbm, idx_hbm, out_hbm, idx_v, rows_v, idx_t, rows_t, sem):
        wid = lax.axis_index("s") * _NC + lax.axis_index("c")
        base = wid * bpw

        def body(j, carry):
            off = base + j * _CH
            pltpu.sync_copy(idx_hbm.at[pl.ds(off, _CH)], idx_v)
            pltpu.async_copy(table_hbm.at[idx_v], rows_v, sem).wait()
            pltpu.sync_copy(rows_v, out_hbm.at[pl.ds(off, _CH)])
            return carry

        lax.fori_loop(0, n_ch, body, 0)
        if tail:
            off = base + n_ch * _CH
            pltpu.sync_copy(idx_hbm.at[pl.ds(off, tail)], idx_t)
            pltpu.async_copy(table_hbm.at[idx_t], rows_t, sem).wait()
            pltpu.sync_copy(rows_t, out_hbm.at[pl.ds(off, tail)])

    return k(table, idx)


def _sc_scatter_add(m, dst, npad, zeros_hbm):
    ecnt, d = m.shape
    bpw = ecnt // _NW
    n_ch = bpw // _CH
    tail = bpw - n_ch * _CH
    rows_per_tile = npad // _NS
    mesh = plsc.VectorSubcoreMesh(core_axis_name="c", subcore_axis_name="s")

    @functools.partial(
        pl.kernel, mesh=mesh,
        out_type=jax.ShapeDtypeStruct((_NC * npad, d), jnp.float32),
        compiler_params=pltpu.CompilerParams(use_tc_tiling_on_sc=False),
        scratch_types=[
            pltpu.VMEM((_CH,), jnp.int32),
            pltpu.VMEM((_CH, d), jnp.float32),
            pltpu.VMEM((max(tail, 1),), jnp.int32),
            pltpu.VMEM((max(tail, 1), d), jnp.float32),
            pltpu.VMEM_SHARED((npad, d), jnp.float32),
        ],
    )
    def k(m_hbm, dst_hbm, z_hbm, out_hbm, idx_v, rows_v, idx_t, rows_t, acc):
        cid = lax.axis_index("c")
        sid = lax.axis_index("s")
        wid = sid * _NC + cid
        base = wid * bpw
        trow = sid * rows_per_tile
        pltpu.sync_copy(z_hbm.at[pl.ds(trow, rows_per_tile)],
                        acc.at[pl.ds(trow, rows_per_tile)])
        plsc.subcore_barrier()

        def body(j, carry):
            off = base + j * _CH
            pltpu.sync_copy(dst_hbm.at[pl.ds(off, _CH)], idx_v)
            pltpu.sync_copy(m_hbm.at[pl.ds(off, _CH)], rows_v)
            pltpu.sync_copy(rows_v, acc.at[idx_v], add=True)
            return carry

        lax.fori_loop(0, n_ch, body, 0)
        if tail:
            off = base + n_ch * _CH
            pltpu.sync_copy(dst_hbm.at[pl.ds(off, tail)], idx_t)
            pltpu.sync_copy(m_hbm.at[pl.ds(off, tail)], rows_t)
            pltpu.sync_copy(rows_t, acc.at[idx_t], add=True)
        plsc.subcore_barrier()
        pltpu.sync_copy(acc.at[pl.ds(trow, rows_per_tile)],
                        out_hbm.at[pl.ds(cid * npad + trow, rows_per_tile)])

    return k(m, dst, zeros_hbm).reshape(_NC, npad, d)


def _pick_block(n, target):
    if n % target == 0:
        return target
    return n


def _proj_kernel(h_ref, wT_ref, b_ref, o_ref):
    o_ref[...] = jax.nn.relu(
        jnp.dot(h_ref[...], wT_ref[...], preferred_element_type=jnp.float32)
        + b_ref[...]
    )


def _project_nodes(h, proj_WT, proj_b):
    n, d_in = h.shape
    d_out = proj_WT.shape[1]
    bn = _pick_block(n, 2000)
    return pl.pallas_call(
        _proj_kernel,
        grid=(n // bn,),
        in_specs=[
            pl.BlockSpec((bn, d_in), lambda i: (i, 0)),
            pl.BlockSpec((d_in, d_out), lambda i: (0, 0)),
            pl.BlockSpec((1, d_out), lambda i: (0, 0)),
        ],
        out_specs=pl.BlockSpec((bn, d_out), lambda i: (i, 0)),
        out_shape=jax.ShapeDtypeStruct((n, d_out), jnp.float32),
    )(h, proj_WT, proj_b.reshape(1, d_out))


def _msg_kernel(eT_ref, hs_ref, w1x_ref, t2_ref, b2mT_ref, m_ref):
    fT = jax.nn.relu(
        jnp.dot(w1x_ref[...], eT_ref[...], preferred_element_type=jnp.float32)
    )
    hsT = hs_ref[...].T
    d_out, b = hsT.shape
    d_eh = fT.shape[0]
    outerT = (hsT[:, None, :] * fT[None, :, :]).reshape(d_out * d_eh, b)
    mT = jnp.dot(t2_ref[...], outerT, preferred_element_type=jnp.float32)
    mT = mT + jnp.dot(b2mT_ref[...], hsT, preferred_element_type=jnp.float32)
    m_ref[...] = mT.T


def _messages(eT, hs, w1x, t2, b2mT):
    d_e1, ecnt = eT.shape
    d_out = hs.shape[1]
    d_eh = w1x.shape[0]
    be = _pick_block(ecnt, 640)
    return pl.pallas_call(
        _msg_kernel,
        grid=(ecnt // be,),
        in_specs=[
            pl.BlockSpec((d_e1, be), lambda i: (0, i)),
            pl.BlockSpec((be, d_out), lambda i: (i, 0)),
            pl.BlockSpec((d_eh, d_e1), lambda i: (0, 0)),
            pl.BlockSpec((d_out, d_out * d_eh), lambda i: (0, 0)),
            pl.BlockSpec((d_out, d_out), lambda i: (0, 0)),
        ],
        out_specs=pl.BlockSpec((be, d_out), lambda i: (i, 0)),
        out_shape=jax.ShapeDtypeStruct((ecnt, d_out), jnp.float32),
    )(eT, hs, w1x, t2, b2mT)


def _gru_kernel(pp_ref, ht_ref, gb_ref,
                wr_i_ref, wz_i_ref, wn_i_ref,
                wr_h_ref, wz_h_ref, wn_h_ref,
                br_i_ref, bz_i_ref, bn_i_ref,
                br_h_ref, bz_h_ref, bn_h_ref,
                o_ref):
    a = jax.nn.relu(pp_ref[0] + pp_ref[1] + gb_ref[...])
    ht = ht_ref[...]
    dot = functools.partial(jnp.dot, preferred_element_type=jnp.float32)
    r = jax.nn.sigmoid(dot(a, wr_i_ref[...]) + br_i_ref[...]
                       + dot(ht, wr_h_ref[...]) + br_h_ref[...])
    z = jax.nn.sigmoid(dot(a, wz_i_ref[...]) + bz_i_ref[...]
                       + dot(ht, wz_h_ref[...]) + bz_h_ref[...])
    nn = jnp.tanh(dot(a, wn_i_ref[...]) + bn_i_ref[...]
                  + r * (dot(ht, wn_h_ref[...]) + bn_h_ref[...]))
    o_ref[...] = (1.0 - z) * nn + z * ht


def _gru_update(partials, ht, gnn_b, gru_ws, gru_bs):
    n, d_out = ht.shape
    bn = _pick_block(n, 2000)
    mat_spec = pl.BlockSpec((d_out, d_out), lambda i: (0, 0))
    vec_spec = pl.BlockSpec((1, d_out), lambda i: (0, 0))
    return pl.pallas_call(
        _gru_kernel,
        grid=(n // bn,),
        in_specs=[
            pl.BlockSpec((partials.shape[0], bn, d_out), lambda i: (0, i, 0)),
            pl.BlockSpec((bn, d_out), lambda i: (i, 0)),
            vec_spec,
            mat_spec, mat_spec, mat_spec,
            mat_spec, mat_spec, mat_spec,
            vec_spec, vec_spec, vec_spec,
            vec_spec, vec_spec, vec_spec,
        ],
        out_specs=pl.BlockSpec((bn, d_out), lambda i: (i, 0)),
        out_shape=jax.ShapeDtypeStruct((n, d_out), jnp.float32),
    )(partials, ht, gnn_b.reshape(1, d_out), *gru_ws, *gru_bs)


def kernel(h, e, edge_index, proj_W, proj_b, edge_W1, edge_b1, edge_W2,
           edge_b2, gnn_b, W_ih, W_hh, b_ih, b_hh):
    n, d_in = h.shape
    ecnt, d_e = e.shape
    d_out = proj_W.shape[0]
    d_eh = edge_W1.shape[0]
    steps = 3

    src = edge_index[0]
    dst = edge_index[1]

    proj_WT = proj_W.T
    eT = jnp.concatenate(
        [e.T, jnp.ones((1, ecnt), jnp.float32)], axis=0)
    w1x = jnp.concatenate([edge_W1, edge_b1[:, None]], axis=1)
    t2 = (edge_W2.reshape(d_out, d_out, d_eh)
          .transpose(1, 0, 2)
          .reshape(d_out, d_out * d_eh))
    b2mT = edge_b2.reshape(d_out, d_out).T
    wih = W_ih.reshape(3, d_out, d_out)
    whh = W_hh.reshape(3, d_out, d_out)
    gru_ws = (wih[0].T, wih[1].T, wih[2].T, whh[0].T, whh[1].T, whh[2].T)
    bih = b_ih.reshape(3, 1, d_out)
    bhh = b_hh.reshape(3, 1, d_out)
    gru_bs = (bih[0], bih[1], bih[2], bhh[0], bhh[1], bhh[2])

    npad = ((n + 16 * _NS - 1) // (16 * _NS)) * (16 * _NS)
    zeros_hbm = jnp.zeros((npad, d_out), jnp.float32)

    hcur = _project_nodes(h, proj_WT, proj_b)
    ht = hcur
    for _ in range(steps):
        hs = _sc_gather(hcur, src)
        m = _messages(eT, hs, w1x, t2, b2mT)
        partials = _sc_scatter_add(m, dst, npad, zeros_hbm)
        hcur = _gru_update(partials, ht, gnn_b, gru_ws, gru_bs)
        ht = hcur
    return hcur
